# baseline (speedup 1.0000x reference)
import functools

import jax
import jax.numpy as jnp
from jax import lax
from jax.experimental import pallas as pl
from jax.experimental.pallas import tpu as pltpu
from jax.experimental.pallas import tpu_sc as plsc

B, S, D = 2, 2048, 1024
H = 2048
E = 8
GH = 512
N = B * S
A = 2 * N
EBLK = 256
CAP = A + E * EBLK
NB = CAP // EBLK
NEG = -1e30

NWORK = 32
TOK_PER_W = N // NWORK
DCH = 64
GCH = 32


GBLK = 256
RBLK = 512


def _gate_body(x_ref, ft_ref, w1_ref, b1_ref, w2_ref, b2_ref, w3_ref,
               b3_ref, temb_ref, wt_ref, bt_ref, gl_ref):
    xb = x_ref[...]
    h = jnp.maximum(jnp.dot(xb, w1_ref[...],
                            preferred_element_type=jnp.float32) + b1_ref[...], 0.0)
    h = jnp.maximum(jnp.dot(h, w2_ref[...],
                            preferred_element_type=jnp.float32) + b2_ref[...], 0.0)
    gl = jnp.dot(h, w3_ref[...], preferred_element_type=jnp.float32) + b3_ref[...]

    tlt = jnp.dot(temb_ref[...], wt_ref[...],
                  preferred_element_type=jnp.float32) + bt_ref[...]
    ft = ft_ref[...]
    for c in range(3):
        gl = gl + jnp.where(ft == c, 1.0, 0.0) * tlt[c:c + 1, :]
    gl_ref[...] = gl


def _route_body(gl_ref, p_ref, wk_ref, bemap_ref, active_ref):
    gl = gl_ref[...]

    lane = jax.lax.broadcasted_iota(jnp.int32, (N, E), 1)
    m1 = jnp.max(gl, axis=-1, keepdims=True)
    i1 = jnp.min(jnp.where(gl == m1, lane, E), axis=-1, keepdims=True)
    gl2 = jnp.where(lane == i1, NEG, gl)
    m2 = jnp.max(gl2, axis=-1, keepdims=True)
    i2 = jnp.min(jnp.where(gl2 == m2, lane, E), axis=-1, keepdims=True)
    e2 = jnp.exp(m2 - m1)
    wk_ref[:, 0:1] = 1.0 / (1.0 + e2)
    wk_ref[:, 1:2] = e2 / (1.0 + e2)

    oh = (jnp.where(lane == i1, 1.0, 0.0)
          + jnp.where(lane == i2, 1.0, 0.0))
    r0 = jax.lax.broadcasted_iota(jnp.int32, (RBLK, RBLK), 0)
    c0 = jax.lax.broadcasted_iota(jnp.int32, (RBLK, RBLK), 1)
    ltri = jnp.where(r0 > c0, 1.0, 0.0)
    pieces = []
    running = jnp.zeros((1, E), jnp.float32)
    for bi in range(N // RBLK):
        xb = oh[bi * RBLK:(bi + 1) * RBLK]
        cb = jnp.dot(ltri, xb, preferred_element_type=jnp.float32)
        pieces.append(cb + running)
        running = running + jnp.sum(xb, axis=0, keepdims=True)
    excl = jnp.concatenate(pieces, axis=0).astype(jnp.int32)
    counts = running.astype(jnp.int32)

    padded = ((counts + (EBLK - 1)) // EBLK) * EBLK
    po = padded
    s = 1
    while s < E:
        po = po + jnp.concatenate(
            [jnp.zeros((1, s), jnp.int32), po[:, :E - s]], axis=1)
        s *= 2
    off = po - padded

    offb = jnp.broadcast_to(off, (N, E))
    r0 = jnp.sum(jnp.where(lane == i1, excl + offb, 0), axis=1, keepdims=True)
    r1 = jnp.sum(jnp.where(lane == i2, excl + offb, 0), axis=1, keepdims=True)
    p_ref[:, 0:1] = r0
    p_ref[:, 1:2] = r1

    starts = jax.lax.broadcasted_iota(jnp.int32, (NB, 1), 0) * EBLK
    cmp = jnp.where(starts >= jnp.broadcast_to(off, (NB, E)), 1, 0)
    be = jnp.sum(cmp, axis=1, keepdims=True) - 1
    lane_nb = jax.lax.broadcasted_iota(jnp.int32, (NB, E), 1)
    ends = jnp.broadcast_to(off + counts, (NB, E))
    sel_end = jnp.sum(jnp.where(lane_nb == be, ends, 0), axis=1, keepdims=True)
    bemap_ref[...] = be
    active_ref[...] = jnp.where(starts < sel_end, 1, 0)


def _full(shape):
    return pl.BlockSpec(shape, lambda *_: tuple(0 for _ in shape))


def _gate_route(x2, ft2, W1, b1, W2, b2, W3, b3, type_emb, Wt, bt):
    gl = pl.pallas_call(
        _gate_body,
        grid=(N // GBLK,),
        in_specs=[
            pl.BlockSpec((GBLK, D), lambda i: (i, 0)),
            pl.BlockSpec((GBLK, 1), lambda i: (i, 0)),
            _full((D, GH)), _full((1, GH)),
            _full((GH, GH // 2)), _full((1, GH // 2)),
            _full((GH // 2, E)), _full((1, E)),
            _full((3, GH // 4)), _full((GH // 4, E)), _full((1, E)),
        ],
        out_specs=pl.BlockSpec((GBLK, E), lambda i: (i, 0)),
        out_shape=jax.ShapeDtypeStruct((N, E), jnp.float32),
    )(x2, ft2, W1, b1, W2, b2, W3, b3, type_emb, Wt, bt)

    return pl.pallas_call(
        _route_body,
        grid=(1,),
        in_specs=[_full((N, E))],
        out_specs=[
            _full((N, 2)), _full((N, 2)), _full((NB, 1)), _full((NB, 1)),
        ],
        out_shape=[
            jax.ShapeDtypeStruct((N, 2), jnp.int32),
            jax.ShapeDtypeStruct((N, 2), jnp.float32),
            jax.ShapeDtypeStruct((NB, 1), jnp.int32),
            jax.ShapeDtypeStruct((NB, 1), jnp.int32),
        ],
    )(gl)


def _sc_scatter_rows(xw, p01):
    mesh = plsc.VectorSubcoreMesh(core_axis_name="c", subcore_axis_name="s")
    DW = D // 2

    @functools.partial(
        pl.kernel, mesh=mesh,
        out_type=jax.ShapeDtypeStruct((CAP, DW), jnp.int32),
        scratch_types=[
            pltpu.VMEM((2, DCH), jnp.int32),
            pltpu.VMEM((DCH, DW), jnp.int32),
            pltpu.SemaphoreType.DMA,
            pltpu.SemaphoreType.DMA,
        ],
    )
    def k(x_hbm, p_hbm, xg_hbm, idx_v, rows_v, sem0, sem1):
        wid = lax.axis_index("s") * 2 + lax.axis_index("c")
        for ci in range(TOK_PER_W // DCH):
            base = wid * TOK_PER_W + ci * DCH
            pltpu.sync_copy(x_hbm.at[pl.ds(base, DCH)], rows_v)
            pltpu.sync_copy(p_hbm.at[0, pl.ds(base, DCH)], idx_v.at[0])
            pltpu.sync_copy(p_hbm.at[1, pl.ds(base, DCH)], idx_v.at[1])
            cp0 = pltpu.async_copy(rows_v, xg_hbm.at[idx_v.at[0]], sem0)
            cp1 = pltpu.async_copy(rows_v, xg_hbm.at[idx_v.at[1]], sem1)
            cp0.wait()
            cp1.wait()

    return k(xw, p01)


def _expert_body(bemap_ref, active_ref, xg_ref, we_ref, be_ref, hid_ref):
    b = pl.program_id(0)

    @pl.when(active_ref[b] == 1)
    def _():
        hid_ref[...] = jnp.maximum(
            jnp.dot(xg_ref[...], we_ref[0],
                    preferred_element_type=jnp.float32) + be_ref[0],
            0.0).astype(jnp.bfloat16)


def _experts(xg, We, be3, bemap, active):
    grid_spec = pltpu.PrefetchScalarGridSpec(
        num_scalar_prefetch=2,
        grid=(NB,),
        in_specs=[
            pl.BlockSpec((EBLK, D), lambda b, bm, ac: (b, 0)),
            pl.BlockSpec((1, D, H), lambda b, bm, ac: (bm[b], 0, 0)),
            pl.BlockSpec((1, 1, H), lambda b, bm, ac: (bm[b], 0, 0)),
        ],
        out_specs=pl.BlockSpec((EBLK, H), lambda b, bm, ac: (b, 0)),
    )
    return pl.pallas_call(
        _expert_body,
        grid_spec=grid_spec,
        out_shape=jax.ShapeDtypeStruct((CAP, H), jnp.bfloat16),
        compiler_params=pltpu.CompilerParams(
            dimension_semantics=("arbitrary",),
        ),
    )(bemap, active, xg, We, be3)


def _sc_gather_rows(hidw, pf):
    mesh = plsc.VectorSubcoreMesh(core_axis_name="c", subcore_axis_name="s")
    nch = A // NWORK // GCH
    HW = H // 2

    @functools.partial(
        pl.kernel, mesh=mesh,
        out_type=jax.ShapeDtypeStruct((A, HW), jnp.int32),
        scratch_types=[
            pltpu.VMEM((nch, GCH), jnp.int32),
            pltpu.VMEM((GCH, HW), jnp.int32),
            pltpu.SemaphoreType.DMA,
        ],
    )
    def k(hid_hbm, pf_hbm, hidg_hbm, idx_v, rows_v, sem):
        wid = lax.axis_index("s") * 2 + lax.axis_index("c")
        for ci in range(nch):
            base = wid * (A // NWORK) + ci * GCH
            pltpu.sync_copy(pf_hbm.at[pl.ds(base, GCH)], idx_v.at[ci])
            pltpu.async_copy(hid_hbm.at[idx_v.at[ci]], rows_v, sem).wait()
            pltpu.sync_copy(rows_v, hidg_hbm.at[pl.ds(base, GCH)])

    return k(hidw, pf)


OBLK = 512


def _proj_body(h3_ref, wk_ref, wo_ref, bo_ref, out_ref):
    w = wk_ref[...]
    comb = (w[:, 0:1] * h3_ref[:, 0, :].astype(jnp.float32)
            + w[:, 1:2] * h3_ref[:, 1, :].astype(jnp.float32))
    out_ref[...] = jnp.dot(comb, wo_ref[...],
                           preferred_element_type=jnp.float32) + bo_ref[...]


def _proj(hidg3, wk, Wo, bo):
    return pl.pallas_call(
        _proj_body,
        grid=(N // OBLK,),
        in_specs=[
            pl.BlockSpec((OBLK, 2, H), lambda i: (i, 0, 0)),
            pl.BlockSpec((OBLK, 2), lambda i: (i, 0)),
            _full((H, D)), _full((1, D)),
        ],
        out_specs=pl.BlockSpec((OBLK, D), lambda i: (i, 0)),
        out_shape=jax.ShapeDtypeStruct((N, D), jnp.float32),
    )(hidg3, wk, Wo, bo)


def _to_words(a):
    return jax.lax.bitcast_convert_type(
        a.reshape(*a.shape[:-1], a.shape[-1] // 2, 2), jnp.int32)


def _from_words(a):
    w = jax.lax.bitcast_convert_type(a, jnp.bfloat16)
    return w.reshape(*a.shape[:-1], a.shape[-1] * 2)


@jax.jit
def _run(x2, ft2, W1, b1, W2, b2, W3, b3, type_emb, Wt, bt, We, be3, Wo, bo):
    p, wk, bemap, active = _gate_route(
        x2, ft2, W1, b1, W2, b2, W3, b3, type_emb, Wt, bt)
    p01 = p.T
    xgw = _sc_scatter_rows(_to_words(x2.astype(jnp.bfloat16)), p01)
    hid = _experts(_from_words(xgw), We, be3,
                   bemap.reshape(NB), active.reshape(NB))
    pf = p.reshape(A)
    hidgw = _sc_gather_rows(_to_words(hid), pf)
    return _proj(_from_words(hidgw).reshape(N, 2, H), wk, Wo, bo)


def kernel(x, feature_types, W1, b1, W2, b2, W3, b3, type_emb, Wt, bt, We, be, Wo, bo):
    x2 = x.reshape(N, D)
    ft2 = feature_types.reshape(N, 1).astype(jnp.int32)
    out = _run(x2, ft2, W1, b1.reshape(1, GH), W2, b2.reshape(1, GH // 2),
               W3, b3.reshape(1, E), type_emb, Wt, bt.reshape(1, E),
               We, be.reshape(E, 1, H), Wo, bo.reshape(1, D))
    return out.reshape(B, S, D)


# speedup vs baseline: 18.6435x; 18.6435x over previous
import functools

import jax
import jax.numpy as jnp
from jax import lax
from jax.experimental import pallas as pl
from jax.experimental.pallas import tpu as pltpu
from jax.experimental.pallas import tpu_sc as plsc

B, S, D = 2, 2048, 1024
H = 2048
E = 8
GH = 512
N = B * S
A = 2 * N
EBLK = 256
CAP = A + E * EBLK
NB = CAP // EBLK
NEG = -1e30
DW = D // 2
HW = H // 2

NWORK = 32
TOK_PER_W = N // NWORK
DCH = 64
GCH = 32

GBLK = 256
NGB = N // GBLK
RBLK = 512


def _pack_halves(hl, hr):
    bl = jax.lax.bitcast_convert_type(hl, jnp.uint32)
    br = jax.lax.bitcast_convert_type(hr, jnp.uint32)
    w = ((bl + 0x8000) >> 16) | (((br + 0x8000) >> 16) << 16)
    return jax.lax.bitcast_convert_type(w, jnp.int32)


def _unpack_halves(wd):
    u = jax.lax.bitcast_convert_type(wd, jnp.uint32)
    lo = jax.lax.bitcast_convert_type(u << 16, jnp.float32)
    hi = jax.lax.bitcast_convert_type(u & jnp.uint32(0xFFFF0000),
                                      jnp.float32)
    return lo, hi


def _gate_route_body(x_ref, ft_ref, w1_ref, b1_ref, w2_ref, b2_ref, w3_ref,
                     b3_ref, temb_ref, wt_ref, bt_ref,
                     xw_ref, p_ref, wk_ref, bemap_ref, active_ref,
                     gl_scr):
    i = pl.program_id(0)

    @pl.when(i < NGB)
    def _gate():
        xb = x_ref[...]
        xw_ref[...] = _pack_halves(xb[:, :DW], xb[:, DW:])
        h = jnp.maximum(jnp.dot(xb, w1_ref[...],
                                preferred_element_type=jnp.float32)
                        + b1_ref[...], 0.0)
        h = jnp.maximum(jnp.dot(h, w2_ref[...],
                                preferred_element_type=jnp.float32)
                        + b2_ref[...], 0.0)
        gl = jnp.dot(h, w3_ref[...],
                     preferred_element_type=jnp.float32) + b3_ref[...]
        tlt = jnp.dot(temb_ref[...], wt_ref[...],
                      preferred_element_type=jnp.float32) + bt_ref[...]
        ft = ft_ref[...]
        for c in range(3):
            gl = gl + jnp.where(ft == c, 1.0, 0.0) * tlt[c:c + 1, :]
        gl_scr[pl.ds(i * GBLK, GBLK), :] = gl

    @pl.when(i == NGB)
    def _route():
        gl = gl_scr[...]

        lane = jax.lax.broadcasted_iota(jnp.int32, (N, E), 1)
        m1 = jnp.max(gl, axis=-1, keepdims=True)
        i1 = jnp.min(jnp.where(gl == m1, lane, E), axis=-1, keepdims=True)
        gl2 = jnp.where(lane == i1, NEG, gl)
        m2 = jnp.max(gl2, axis=-1, keepdims=True)
        i2 = jnp.min(jnp.where(gl2 == m2, lane, E), axis=-1, keepdims=True)
        e2 = jnp.exp(m2 - m1)
        wk_ref[:, 0:1] = 1.0 / (1.0 + e2)
        wk_ref[:, 1:2] = e2 / (1.0 + e2)

        oh = (jnp.where(lane == i1, 1.0, 0.0)
              + jnp.where(lane == i2, 1.0, 0.0))
        r0 = jax.lax.broadcasted_iota(jnp.int32, (RBLK, RBLK), 0)
        c0 = jax.lax.broadcasted_iota(jnp.int32, (RBLK, RBLK), 1)
        ltri = jnp.where(r0 > c0, 1.0, 0.0)
        pieces = []
        running = jnp.zeros((1, E), jnp.float32)
        for bi in range(N // RBLK):
            xb = oh[bi * RBLK:(bi + 1) * RBLK]
            cb = jnp.dot(ltri, xb, preferred_element_type=jnp.float32)
            pieces.append(cb + running)
            running = running + jnp.sum(xb, axis=0, keepdims=True)
        excl = jnp.concatenate(pieces, axis=0).astype(jnp.int32)
        counts = running.astype(jnp.int32)

        padded = ((counts + (EBLK - 1)) // EBLK) * EBLK
        po = padded
        s = 1
        while s < E:
            po = po + jnp.concatenate(
                [jnp.zeros((1, s), jnp.int32), po[:, :E - s]], axis=1)
            s *= 2
        off = po - padded

        offb = jnp.broadcast_to(off, (N, E))
        p_ref[:, 0:1] = jnp.sum(jnp.where(lane == i1, excl + offb, 0),
                                axis=1, keepdims=True)
        p_ref[:, 1:2] = jnp.sum(jnp.where(lane == i2, excl + offb, 0),
                                axis=1, keepdims=True)

        starts = jax.lax.broadcasted_iota(jnp.int32, (NB, 1), 0) * EBLK
        cmp = jnp.where(starts >= jnp.broadcast_to(off, (NB, E)), 1, 0)
        be = jnp.sum(cmp, axis=1, keepdims=True) - 1
        lane_nb = jax.lax.broadcasted_iota(jnp.int32, (NB, E), 1)
        ends = jnp.broadcast_to(off + counts, (NB, E))
        sel_end = jnp.sum(jnp.where(lane_nb == be, ends, 0),
                          axis=1, keepdims=True)
        bemap_ref[...] = be
        active_ref[...] = jnp.where(starts < sel_end, 1, 0)


def _full(shape):
    return pl.BlockSpec(shape, lambda *_: tuple(0 for _ in shape))


def _gate_route(x2, ft2, W1, b1, W2, b2, W3, b3, type_emb, Wt, bt):
    blk = lambda i: (jnp.minimum(i, NGB - 1), 0)
    return pl.pallas_call(
        _gate_route_body,
        grid=(NGB + 1,),
        in_specs=[
            pl.BlockSpec((GBLK, D), blk),
            pl.BlockSpec((GBLK, 1), blk),
            _full((D, GH)), _full((1, GH)),
            _full((GH, GH // 2)), _full((1, GH // 2)),
            _full((GH // 2, E)), _full((1, E)),
            _full((3, GH // 4)), _full((GH // 4, E)), _full((1, E)),
        ],
        out_specs=[
            pl.BlockSpec((GBLK, DW), blk),
            _full((N, 2)), _full((N, 2)), _full((NB, 1)), _full((NB, 1)),
        ],
        out_shape=[
            jax.ShapeDtypeStruct((N, DW), jnp.int32),
            jax.ShapeDtypeStruct((N, 2), jnp.int32),
            jax.ShapeDtypeStruct((N, 2), jnp.float32),
            jax.ShapeDtypeStruct((NB, 1), jnp.int32),
            jax.ShapeDtypeStruct((NB, 1), jnp.int32),
        ],
        scratch_shapes=[pltpu.VMEM((N, E), jnp.float32)],
        compiler_params=pltpu.CompilerParams(
            dimension_semantics=("arbitrary",),
        ),
    )(x2, ft2, W1, b1, W2, b2, W3, b3, type_emb, Wt, bt)


def _sc_scatter_rows(xw, p01):
    mesh = plsc.VectorSubcoreMesh(core_axis_name="c", subcore_axis_name="s")

    @functools.partial(
        pl.kernel, mesh=mesh,
        out_type=jax.ShapeDtypeStruct((CAP, DW), jnp.int32),
        scratch_types=[
            pltpu.VMEM((2, DCH), jnp.int32),
            pltpu.VMEM((DCH, DW), jnp.int32),
            pltpu.SemaphoreType.DMA,
            pltpu.SemaphoreType.DMA,
        ],
    )
    def k(x_hbm, p_hbm, xg_hbm, idx_v, rows_v, sem0, sem1):
        wid = lax.axis_index("s") * 2 + lax.axis_index("c")
        for ci in range(TOK_PER_W // DCH):
            base = wid * TOK_PER_W + ci * DCH
            pltpu.sync_copy(x_hbm.at[pl.ds(base, DCH)], rows_v)
            pltpu.sync_copy(p_hbm.at[0, pl.ds(base, DCH)], idx_v.at[0])
            pltpu.sync_copy(p_hbm.at[1, pl.ds(base, DCH)], idx_v.at[1])
            cp0 = pltpu.async_copy(rows_v, xg_hbm.at[idx_v.at[0]], sem0)
            cp1 = pltpu.async_copy(rows_v, xg_hbm.at[idx_v.at[1]], sem1)
            cp0.wait()
            cp1.wait()

    return k(xw, p01)


def _expert_body(bemap_ref, active_ref, xg_ref, we_ref, be_ref, hid_ref):
    b = pl.program_id(0)

    @pl.when(active_ref[b] == 1)
    def _():
        xl, xr = _unpack_halves(xg_ref[...])
        we = we_ref[0]
        h = jnp.maximum(
            jnp.dot(xl, we[:DW, :], preferred_element_type=jnp.float32)
            + jnp.dot(xr, we[DW:, :], preferred_element_type=jnp.float32)
            + be_ref[0], 0.0)
        hid_ref[...] = _pack_halves(h[:, :HW], h[:, HW:])


def _experts(xgw, We, be3, bemap, active):
    grid_spec = pltpu.PrefetchScalarGridSpec(
        num_scalar_prefetch=2,
        grid=(NB,),
        in_specs=[
            pl.BlockSpec((EBLK, DW), lambda b, bm, ac: (b, 0)),
            pl.BlockSpec((1, D, H), lambda b, bm, ac: (bm[b], 0, 0)),
            pl.BlockSpec((1, 1, H), lambda b, bm, ac: (bm[b], 0, 0)),
        ],
        out_specs=pl.BlockSpec((EBLK, HW), lambda b, bm, ac: (b, 0)),
    )
    return pl.pallas_call(
        _expert_body,
        grid_spec=grid_spec,
        out_shape=jax.ShapeDtypeStruct((CAP, HW), jnp.int32),
        compiler_params=pltpu.CompilerParams(
            dimension_semantics=("arbitrary",),
        ),
    )(bemap, active, xgw, We, be3)


def _sc_gather_rows(hidw, pf):
    mesh = plsc.VectorSubcoreMesh(core_axis_name="c", subcore_axis_name="s")
    nch = A // NWORK // GCH

    @functools.partial(
        pl.kernel, mesh=mesh,
        out_type=jax.ShapeDtypeStruct((A, HW), jnp.int32),
        scratch_types=[
            pltpu.VMEM((nch, GCH), jnp.int32),
            pltpu.VMEM((GCH, HW), jnp.int32),
            pltpu.SemaphoreType.DMA,
        ],
    )
    def k(hid_hbm, pf_hbm, hidg_hbm, idx_v, rows_v, sem):
        wid = lax.axis_index("s") * 2 + lax.axis_index("c")
        for ci in range(nch):
            base = wid * (A // NWORK) + ci * GCH
            pltpu.sync_copy(pf_hbm.at[pl.ds(base, GCH)], idx_v.at[ci])
            pltpu.async_copy(hid_hbm.at[idx_v.at[ci]], rows_v, sem).wait()
            pltpu.sync_copy(rows_v, hidg_hbm.at[pl.ds(base, GCH)])

    return k(hidw, pf)


OBLK = 512


def _proj_body(h3_ref, wk_ref, wo_ref, bo_ref, out_ref):
    w = wk_ref[...]
    h0l, h0r = _unpack_halves(h3_ref[:, 0, :])
    h1l, h1r = _unpack_halves(h3_ref[:, 1, :])
    comb_l = w[:, 0:1] * h0l + w[:, 1:2] * h1l
    comb_r = w[:, 0:1] * h0r + w[:, 1:2] * h1r
    wo = wo_ref[...]
    out_ref[...] = (
        jnp.dot(comb_l, wo[:HW, :], preferred_element_type=jnp.float32)
        + jnp.dot(comb_r, wo[HW:, :], preferred_element_type=jnp.float32)
        + bo_ref[...])


def _proj(hidg3, wk, Wo, bo):
    return pl.pallas_call(
        _proj_body,
        grid=(N // OBLK,),
        in_specs=[
            pl.BlockSpec((OBLK, 2, HW), lambda i: (i, 0, 0)),
            pl.BlockSpec((OBLK, 2), lambda i: (i, 0)),
            _full((H, D)), _full((1, D)),
        ],
        out_specs=pl.BlockSpec((OBLK, D), lambda i: (i, 0)),
        out_shape=jax.ShapeDtypeStruct((N, D), jnp.float32),
    )(hidg3, wk, Wo, bo)


@jax.jit
def _run(x2, ft2, W1, b1, W2, b2, W3, b3, type_emb, Wt, bt, We, be3, Wo, bo):
    xw, p, wk, bemap, active = _gate_route(
        x2, ft2, W1, b1, W2, b2, W3, b3, type_emb, Wt, bt)
    p01 = p.T
    xgw = _sc_scatter_rows(xw, p01)
    hidw = _experts(xgw, We, be3, bemap.reshape(NB), active.reshape(NB))
    pf = p.reshape(A)
    hidgw = _sc_gather_rows(hidw, pf)
    return _proj(hidgw.reshape(N, 2, HW), wk, Wo, bo)


def kernel(x, feature_types, W1, b1, W2, b2, W3, b3, type_emb, Wt, bt, We, be, Wo, bo):
    x2 = x.reshape(N, D)
    ft2 = feature_types.reshape(N, 1).astype(jnp.int32)
    out = _run(x2, ft2, W1, b1.reshape(1, GH), W2, b2.reshape(1, GH // 2),
               W3, b3.reshape(1, E), type_emb, Wt, bt.reshape(1, E),
               We, be.reshape(E, 1, H), Wo, bo.reshape(1, D))
    return out.reshape(B, S, D)


# speedup vs baseline: 25.5347x; 1.3696x over previous
import functools

import jax
import jax.numpy as jnp
from jax import lax
from jax.experimental import pallas as pl
from jax.experimental.pallas import tpu as pltpu
from jax.experimental.pallas import tpu_sc as plsc

B, S, D = 2, 2048, 1024
H = 2048
E = 8
GH = 512
N = B * S
A = 2 * N
EBLK = 256
CAP = A + E * EBLK
NB = CAP // EBLK
NEG = -1e30
DW = D // 2
HW = H // 2

NWORK = 32
TOK_PER_W = N // NWORK
DCH = 64
GCH = 32

GBLK = 256
NGB = N // GBLK
RBLK = 512


def _pack_halves(hl, hr):
    bl = jax.lax.bitcast_convert_type(hl, jnp.uint32)
    br = jax.lax.bitcast_convert_type(hr, jnp.uint32)
    w = ((bl + 0x8000) >> 16) | (((br + 0x8000) >> 16) << 16)
    return jax.lax.bitcast_convert_type(w, jnp.int32)


def _unpack_halves(wd):
    u = jax.lax.bitcast_convert_type(wd, jnp.uint32)
    lo = jax.lax.bitcast_convert_type(u << 16, jnp.float32)
    hi = jax.lax.bitcast_convert_type(u & jnp.uint32(0xFFFF0000),
                                      jnp.float32)
    return lo, hi


def _gate_route_body(x_ref, ft_ref, w1_ref, b1_ref, w2_ref, b2_ref, w3_ref,
                     b3_ref, temb_ref, wt_ref, bt_ref,
                     xw_ref, p_ref, wk_ref, bemap_ref, active_ref,
                     gl_scr):
    i = pl.program_id(0)

    @pl.when(i < NGB)
    def _gate():
        xb = x_ref[...]
        xw_ref[...] = _pack_halves(xb[:, :DW], xb[:, DW:])
        h = jnp.maximum(jnp.dot(xb, w1_ref[...],
                                preferred_element_type=jnp.float32)
                        + b1_ref[...], 0.0)
        h = jnp.maximum(jnp.dot(h, w2_ref[...],
                                preferred_element_type=jnp.float32)
                        + b2_ref[...], 0.0)
        gl = jnp.dot(h, w3_ref[...],
                     preferred_element_type=jnp.float32) + b3_ref[...]
        tlt = jnp.dot(temb_ref[...], wt_ref[...],
                      preferred_element_type=jnp.float32) + bt_ref[...]
        ft = ft_ref[...]
        for c in range(3):
            gl = gl + jnp.where(ft == c, 1.0, 0.0) * tlt[c:c + 1, :]
        gl_scr[pl.ds(i * GBLK, GBLK), :] = gl

    @pl.when(i == NGB)
    def _route():
        gl = gl_scr[...]

        lane = jax.lax.broadcasted_iota(jnp.int32, (N, E), 1)
        m1 = jnp.max(gl, axis=-1, keepdims=True)
        i1 = jnp.min(jnp.where(gl == m1, lane, E), axis=-1, keepdims=True)
        gl2 = jnp.where(lane == i1, NEG, gl)
        m2 = jnp.max(gl2, axis=-1, keepdims=True)
        i2 = jnp.min(jnp.where(gl2 == m2, lane, E), axis=-1, keepdims=True)
        e2 = jnp.exp(m2 - m1)
        wk_ref[:, 0:1] = 1.0 / (1.0 + e2)
        wk_ref[:, 1:2] = e2 / (1.0 + e2)

        oh = (jnp.where(lane == i1, 1.0, 0.0)
              + jnp.where(lane == i2, 1.0, 0.0))
        r0 = jax.lax.broadcasted_iota(jnp.int32, (RBLK, RBLK), 0)
        c0 = jax.lax.broadcasted_iota(jnp.int32, (RBLK, RBLK), 1)
        ltri = jnp.where(r0 > c0, 1.0, 0.0)
        pieces = []
        running = jnp.zeros((1, E), jnp.float32)
        for bi in range(N // RBLK):
            xb = oh[bi * RBLK:(bi + 1) * RBLK]
            cb = jnp.dot(ltri, xb, preferred_element_type=jnp.float32)
            pieces.append(cb + running)
            running = running + jnp.sum(xb, axis=0, keepdims=True)
        excl = jnp.concatenate(pieces, axis=0).astype(jnp.int32)
        counts = running.astype(jnp.int32)

        padded = ((counts + (EBLK - 1)) // EBLK) * EBLK
        po = padded
        s = 1
        while s < E:
            po = po + jnp.concatenate(
                [jnp.zeros((1, s), jnp.int32), po[:, :E - s]], axis=1)
            s *= 2
        off = po - padded

        offb = jnp.broadcast_to(off, (N, E))
        p_ref[:, 0:1] = jnp.sum(jnp.where(lane == i1, excl + offb, 0),
                                axis=1, keepdims=True)
        p_ref[:, 1:2] = jnp.sum(jnp.where(lane == i2, excl + offb, 0),
                                axis=1, keepdims=True)

        starts = jax.lax.broadcasted_iota(jnp.int32, (NB, 1), 0) * EBLK
        cmp = jnp.where(starts >= jnp.broadcast_to(off, (NB, E)), 1, 0)
        be = jnp.sum(cmp, axis=1, keepdims=True) - 1
        lane_nb = jax.lax.broadcasted_iota(jnp.int32, (NB, E), 1)
        ends = jnp.broadcast_to(off + counts, (NB, E))
        sel_end = jnp.sum(jnp.where(lane_nb == be, ends, 0),
                          axis=1, keepdims=True)
        bemap_ref[...] = be
        active_ref[...] = jnp.where(starts < sel_end, 1, 0)


def _full(shape):
    return pl.BlockSpec(shape, lambda *_: tuple(0 for _ in shape))


def _gate_route(x2, ft2, W1, b1, W2, b2, W3, b3, type_emb, Wt, bt):
    blk = lambda i: (jnp.minimum(i, NGB - 1), 0)
    return pl.pallas_call(
        _gate_route_body,
        grid=(NGB + 1,),
        in_specs=[
            pl.BlockSpec((GBLK, D), blk),
            pl.BlockSpec((GBLK, 1), blk),
            _full((D, GH)), _full((1, GH)),
            _full((GH, GH // 2)), _full((1, GH // 2)),
            _full((GH // 2, E)), _full((1, E)),
            _full((3, GH // 4)), _full((GH // 4, E)), _full((1, E)),
        ],
        out_specs=[
            pl.BlockSpec((GBLK, DW), blk),
            _full((N, 2)), _full((N, 2)), _full((NB, 1)), _full((NB, 1)),
        ],
        out_shape=[
            jax.ShapeDtypeStruct((N, DW), jnp.int32),
            jax.ShapeDtypeStruct((N, 2), jnp.int32),
            jax.ShapeDtypeStruct((N, 2), jnp.float32),
            jax.ShapeDtypeStruct((NB, 1), jnp.int32),
            jax.ShapeDtypeStruct((NB, 1), jnp.int32),
        ],
        scratch_shapes=[pltpu.VMEM((N, E), jnp.float32)],
        compiler_params=pltpu.CompilerParams(
            dimension_semantics=("arbitrary",),
        ),
    )(x2, ft2, W1, b1, W2, b2, W3, b3, type_emb, Wt, bt)


def _sc_scatter_rows(xw, p01):
    mesh = plsc.VectorSubcoreMesh(core_axis_name="c", subcore_axis_name="s")

    @functools.partial(
        pl.kernel, mesh=mesh,
        out_type=jax.ShapeDtypeStruct((CAP, DW), jnp.int32),
        scratch_types=[
            pltpu.VMEM((2, DCH), jnp.int32),
            pltpu.VMEM((DCH, DW), jnp.int32),
            pltpu.SemaphoreType.DMA,
            pltpu.SemaphoreType.DMA,
        ],
    )
    def k(x_hbm, p_hbm, xg_hbm, idx_v, rows_v, sem0, sem1):
        wid = lax.axis_index("s") * 2 + lax.axis_index("c")
        for ci in range(TOK_PER_W // DCH):
            base = wid * TOK_PER_W + ci * DCH
            pltpu.sync_copy(x_hbm.at[pl.ds(base, DCH)], rows_v)
            pltpu.sync_copy(p_hbm.at[0, pl.ds(base, DCH)], idx_v.at[0])
            pltpu.sync_copy(p_hbm.at[1, pl.ds(base, DCH)], idx_v.at[1])
            cp0 = pltpu.async_copy(rows_v, xg_hbm.at[idx_v.at[0]], sem0)
            cp1 = pltpu.async_copy(rows_v, xg_hbm.at[idx_v.at[1]], sem1)
            cp0.wait()
            cp1.wait()

    return k(xw, p01)


def _expert_body(bemap_ref, active_ref, xg_ref, we_ref, be_ref, hid_ref):
    b = pl.program_id(0)

    @pl.when(active_ref[b] == 1)
    def _():
        xl, xr = _unpack_halves(xg_ref[...])
        we = we_ref[0]
        h = jnp.maximum(
            jnp.dot(xl, we[:DW, :], preferred_element_type=jnp.float32)
            + jnp.dot(xr, we[DW:, :], preferred_element_type=jnp.float32)
            + be_ref[0], 0.0)
        hid_ref[...] = _pack_halves(h[:, :HW], h[:, HW:])


def _experts(xgw, We, be3, bemap, active):
    grid_spec = pltpu.PrefetchScalarGridSpec(
        num_scalar_prefetch=2,
        grid=(NB,),
        in_specs=[
            pl.BlockSpec((EBLK, DW), lambda b, bm, ac: (b, 0)),
            pl.BlockSpec((1, D, H), lambda b, bm, ac: (bm[b], 0, 0)),
            pl.BlockSpec((1, 1, H), lambda b, bm, ac: (bm[b], 0, 0)),
        ],
        out_specs=pl.BlockSpec((EBLK, HW), lambda b, bm, ac: (b, 0)),
    )
    return pl.pallas_call(
        _expert_body,
        grid_spec=grid_spec,
        out_shape=jax.ShapeDtypeStruct((CAP, HW), jnp.int32),
        compiler_params=pltpu.CompilerParams(
            dimension_semantics=("arbitrary",),
        ),
    )(bemap, active, xgw, We, be3)


def _sc_gather_rows(hidw, pf):
    mesh = plsc.VectorSubcoreMesh(core_axis_name="c", subcore_axis_name="s")
    nch = A // NWORK // GCH

    @functools.partial(
        pl.kernel, mesh=mesh,
        out_type=jax.ShapeDtypeStruct((A, HW), jnp.int32),
        scratch_types=[
            pltpu.VMEM((nch, GCH), jnp.int32),
            pltpu.VMEM((GCH, HW), jnp.int32),
            pltpu.SemaphoreType.DMA,
        ],
    )
    def k(hid_hbm, pf_hbm, hidg_hbm, idx_v, rows_v, sem):
        wid = lax.axis_index("s") * 2 + lax.axis_index("c")
        for ci in range(nch):
            base = wid * (A // NWORK) + ci * GCH
            pltpu.sync_copy(pf_hbm.at[pl.ds(base, GCH)], idx_v.at[ci])
            pltpu.async_copy(hid_hbm.at[idx_v.at[ci]], rows_v, sem).wait()
            pltpu.sync_copy(rows_v, hidg_hbm.at[pl.ds(base, GCH)])

    return k(hidw, pf)


OBLK = 512


def _proj_body(h0_ref, h1_ref, wk_ref, wo_ref, bo_ref, out_ref):
    w = wk_ref[...]
    h0l, h0r = _unpack_halves(h0_ref[...])
    h1l, h1r = _unpack_halves(h1_ref[...])
    comb_l = w[:, 0:1] * h0l + w[:, 1:2] * h1l
    comb_r = w[:, 0:1] * h0r + w[:, 1:2] * h1r
    wo = wo_ref[...]
    out_ref[...] = (
        jnp.dot(comb_l, wo[:HW, :], preferred_element_type=jnp.float32)
        + jnp.dot(comb_r, wo[HW:, :], preferred_element_type=jnp.float32)
        + bo_ref[...])


def _proj(hidgw, wk, Wo, bo):
    return pl.pallas_call(
        _proj_body,
        grid=(N // OBLK,),
        in_specs=[
            pl.BlockSpec((OBLK, HW), lambda i: (i, 0)),
            pl.BlockSpec((OBLK, HW), lambda i: (i + N // OBLK, 0)),
            pl.BlockSpec((OBLK, 2), lambda i: (i, 0)),
            _full((H, D)), _full((1, D)),
        ],
        out_specs=pl.BlockSpec((OBLK, D), lambda i: (i, 0)),
        out_shape=jax.ShapeDtypeStruct((N, D), jnp.float32),
    )(hidgw, hidgw, wk, Wo, bo)


@jax.jit
def _run(x2, ft2, W1, b1, W2, b2, W3, b3, type_emb, Wt, bt, We, be3, Wo, bo):
    xw, p, wk, bemap, active = _gate_route(
        x2, ft2, W1, b1, W2, b2, W3, b3, type_emb, Wt, bt)
    p01 = p.T
    xgw = _sc_scatter_rows(xw, p01)
    hidw = _experts(xgw, We, be3, bemap.reshape(NB), active.reshape(NB))
    pf = p01.reshape(A)
    hidgw = _sc_gather_rows(hidw, pf)
    return _proj(hidgw, wk, Wo, bo)


def kernel(x, feature_types, W1, b1, W2, b2, W3, b3, type_emb, Wt, bt, We, be, Wo, bo):
    x2 = x.reshape(N, D)
    ft2 = feature_types.reshape(N, 1).astype(jnp.int32)
    out = _run(x2, ft2, W1, b1.reshape(1, GH), W2, b2.reshape(1, GH // 2),
               W3, b3.reshape(1, E), type_emb, Wt, bt.reshape(1, E),
               We, be.reshape(E, 1, H), Wo, bo.reshape(1, D))
    return out.reshape(B, S, D)


# speedup vs baseline: 26.3433x; 1.0317x over previous
import functools

import jax
import jax.numpy as jnp
from jax import lax
from jax.experimental import pallas as pl
from jax.experimental.pallas import tpu as pltpu
from jax.experimental.pallas import tpu_sc as plsc

B, S, D = 2, 2048, 1024
H = 2048
E = 8
GH = 512
N = B * S
A = 2 * N
EBLK = 256
CAP = A + E * EBLK
NB = CAP // EBLK
NEG = -1e30
DW = D // 2
HW = H // 2

NWORK = 32
TOK_PER_W = N // NWORK
DCH = 64
GCH = 32

GBLK = 256
NGB = N // GBLK
RBLK = 512


def _pack_halves(hl, hr):
    bl = jax.lax.bitcast_convert_type(hl, jnp.uint32)
    br = jax.lax.bitcast_convert_type(hr, jnp.uint32)
    w = ((bl + 0x8000) >> 16) | (((br + 0x8000) >> 16) << 16)
    return jax.lax.bitcast_convert_type(w, jnp.int32)


def _unpack_halves(wd):
    u = jax.lax.bitcast_convert_type(wd, jnp.uint32)
    lo = jax.lax.bitcast_convert_type(u << 16, jnp.float32)
    hi = jax.lax.bitcast_convert_type(u & jnp.uint32(0xFFFF0000),
                                      jnp.float32)
    return lo, hi


def _gate_route_body(x_ref, ft_ref, w1_ref, b1_ref, w2_ref, b2_ref, w3_ref,
                     b3_ref, temb_ref, wt_ref, bt_ref,
                     xw_ref, p_ref, wk_ref, bemap_ref, active_ref,
                     gl_scr):
    i = pl.program_id(0)

    @pl.when(i < NGB)
    def _gate():
        xb = x_ref[...]
        xw_ref[...] = _pack_halves(xb[:, :DW], xb[:, DW:])
        h = jnp.maximum(jnp.dot(xb, w1_ref[...],
                                preferred_element_type=jnp.float32)
                        + b1_ref[...], 0.0)
        h = jnp.maximum(jnp.dot(h, w2_ref[...],
                                preferred_element_type=jnp.float32)
                        + b2_ref[...], 0.0)
        gl = jnp.dot(h, w3_ref[...],
                     preferred_element_type=jnp.float32) + b3_ref[...]
        tlt = jnp.dot(temb_ref[...], wt_ref[...],
                      preferred_element_type=jnp.float32) + bt_ref[...]
        ft = ft_ref[...]
        for c in range(3):
            gl = gl + jnp.where(ft == c, 1.0, 0.0) * tlt[c:c + 1, :]
        gl_scr[pl.ds(i * GBLK, GBLK), :] = gl

    @pl.when(i == NGB)
    def _route():
        gl = gl_scr[...]

        lane = jax.lax.broadcasted_iota(jnp.int32, (N, E), 1)
        m1 = jnp.max(gl, axis=-1, keepdims=True)
        i1 = jnp.min(jnp.where(gl == m1, lane, E), axis=-1, keepdims=True)
        gl2 = jnp.where(lane == i1, NEG, gl)
        m2 = jnp.max(gl2, axis=-1, keepdims=True)
        i2 = jnp.min(jnp.where(gl2 == m2, lane, E), axis=-1, keepdims=True)
        e2 = jnp.exp(m2 - m1)
        wk_ref[:, 0:1] = 1.0 / (1.0 + e2)
        wk_ref[:, 1:2] = e2 / (1.0 + e2)

        oh = (jnp.where(lane == i1, 1.0, 0.0)
              + jnp.where(lane == i2, 1.0, 0.0))
        r0 = jax.lax.broadcasted_iota(jnp.int32, (RBLK, RBLK), 0)
        c0 = jax.lax.broadcasted_iota(jnp.int32, (RBLK, RBLK), 1)
        ltri = jnp.where(r0 > c0, 1.0, 0.0)
        pieces = []
        running = jnp.zeros((1, E), jnp.float32)
        for bi in range(N // RBLK):
            xb = oh[bi * RBLK:(bi + 1) * RBLK]
            cb = jnp.dot(ltri, xb, preferred_element_type=jnp.float32)
            pieces.append(cb + running)
            running = running + jnp.sum(xb, axis=0, keepdims=True)
        excl = jnp.concatenate(pieces, axis=0).astype(jnp.int32)
        counts = running.astype(jnp.int32)

        padded = ((counts + (EBLK - 1)) // EBLK) * EBLK
        po = padded
        s = 1
        while s < E:
            po = po + jnp.concatenate(
                [jnp.zeros((1, s), jnp.int32), po[:, :E - s]], axis=1)
            s *= 2
        off = po - padded

        offb = jnp.broadcast_to(off, (N, E))
        p_ref[:, 0:1] = jnp.sum(jnp.where(lane == i1, excl + offb, 0),
                                axis=1, keepdims=True)
        p_ref[:, 1:2] = jnp.sum(jnp.where(lane == i2, excl + offb, 0),
                                axis=1, keepdims=True)

        starts = jax.lax.broadcasted_iota(jnp.int32, (NB, 1), 0) * EBLK
        cmp = jnp.where(starts >= jnp.broadcast_to(off, (NB, E)), 1, 0)
        be = jnp.sum(cmp, axis=1, keepdims=True) - 1
        lane_nb = jax.lax.broadcasted_iota(jnp.int32, (NB, E), 1)
        ends = jnp.broadcast_to(off + counts, (NB, E))
        sel_end = jnp.sum(jnp.where(lane_nb == be, ends, 0),
                          axis=1, keepdims=True)
        bemap_ref[...] = be
        active_ref[...] = jnp.where(starts < sel_end, 1, 0)


def _full(shape):
    return pl.BlockSpec(shape, lambda *_: tuple(0 for _ in shape))


def _gate_route(x2, ft2, W1, b1, W2, b2, W3, b3, type_emb, Wt, bt):
    blk = lambda i: (jnp.minimum(i, NGB - 1), 0)
    return pl.pallas_call(
        _gate_route_body,
        grid=(NGB + 1,),
        in_specs=[
            pl.BlockSpec((GBLK, D), blk),
            pl.BlockSpec((GBLK, 1), blk),
            _full((D, GH)), _full((1, GH)),
            _full((GH, GH // 2)), _full((1, GH // 2)),
            _full((GH // 2, E)), _full((1, E)),
            _full((3, GH // 4)), _full((GH // 4, E)), _full((1, E)),
        ],
        out_specs=[
            pl.BlockSpec((GBLK, DW), blk),
            _full((N, 2)), _full((N, 2)), _full((NB, 1)), _full((NB, 1)),
        ],
        out_shape=[
            jax.ShapeDtypeStruct((N, DW), jnp.int32),
            jax.ShapeDtypeStruct((N, 2), jnp.int32),
            jax.ShapeDtypeStruct((N, 2), jnp.float32),
            jax.ShapeDtypeStruct((NB, 1), jnp.int32),
            jax.ShapeDtypeStruct((NB, 1), jnp.int32),
        ],
        scratch_shapes=[pltpu.VMEM((N, E), jnp.float32)],
        compiler_params=pltpu.CompilerParams(
            dimension_semantics=("arbitrary",),
        ),
    )(x2, ft2, W1, b1, W2, b2, W3, b3, type_emb, Wt, bt)


def _sc_scatter_rows(xw, p01):
    mesh = plsc.VectorSubcoreMesh(core_axis_name="c", subcore_axis_name="s")

    @functools.partial(
        pl.kernel, mesh=mesh,
        out_type=jax.ShapeDtypeStruct((CAP, DW), jnp.int32),
        scratch_types=[
            pltpu.VMEM((2, DCH), jnp.int32),
            pltpu.VMEM((DCH, DW), jnp.int32),
            pltpu.SemaphoreType.DMA,
            pltpu.SemaphoreType.DMA,
        ],
    )
    def k(x_hbm, p_hbm, xg_hbm, idx_v, rows_v, sem0, sem1):
        wid = lax.axis_index("s") * 2 + lax.axis_index("c")
        for ci in range(TOK_PER_W // DCH):
            base = wid * TOK_PER_W + ci * DCH
            pltpu.sync_copy(x_hbm.at[pl.ds(base, DCH)], rows_v)
            pltpu.sync_copy(p_hbm.at[0, pl.ds(base, DCH)], idx_v.at[0])
            pltpu.sync_copy(p_hbm.at[1, pl.ds(base, DCH)], idx_v.at[1])
            cp0 = pltpu.async_copy(rows_v, xg_hbm.at[idx_v.at[0]], sem0)
            cp1 = pltpu.async_copy(rows_v, xg_hbm.at[idx_v.at[1]], sem1)
            cp0.wait()
            cp1.wait()

    return k(xw, p01)


def _expert_body(bemap_ref, active_ref, xg_ref, we_ref, be_ref, hid_ref):
    b = pl.program_id(0)

    @pl.when(active_ref[b] == 1)
    def _():
        xl, xr = _unpack_halves(xg_ref[...])
        we = we_ref[0]
        h = jnp.maximum(
            jnp.dot(xl, we[:DW, :], preferred_element_type=jnp.float32)
            + jnp.dot(xr, we[DW:, :], preferred_element_type=jnp.float32)
            + be_ref[0], 0.0)
        hid_ref[...] = _pack_halves(h[:, :HW], h[:, HW:])


def _experts(xgw, We, be3, bemap, active):
    grid_spec = pltpu.PrefetchScalarGridSpec(
        num_scalar_prefetch=2,
        grid=(NB,),
        in_specs=[
            pl.BlockSpec((EBLK, DW), lambda b, bm, ac: (b, 0)),
            pl.BlockSpec((1, D, H), lambda b, bm, ac: (bm[b], 0, 0)),
            pl.BlockSpec((1, 1, H), lambda b, bm, ac: (bm[b], 0, 0)),
        ],
        out_specs=pl.BlockSpec((EBLK, HW), lambda b, bm, ac: (b, 0)),
    )
    return pl.pallas_call(
        _expert_body,
        grid_spec=grid_spec,
        out_shape=jax.ShapeDtypeStruct((CAP, HW), jnp.int32),
        compiler_params=pltpu.CompilerParams(
            dimension_semantics=("arbitrary",),
        ),
    )(bemap, active, xgw, We, be3)


def _sc_gather_rows(hidw, pf):
    mesh = plsc.VectorSubcoreMesh(core_axis_name="c", subcore_axis_name="s")
    nch = A // NWORK // GCH

    @functools.partial(
        pl.kernel, mesh=mesh,
        out_type=jax.ShapeDtypeStruct((A, HW), jnp.int32),
        scratch_types=[
            pltpu.VMEM((nch, GCH), jnp.int32),
            pltpu.VMEM((GCH, HW), jnp.int32),
            pltpu.VMEM((GCH, HW), jnp.int32),
            pltpu.SemaphoreType.DMA,
            pltpu.SemaphoreType.DMA,
        ],
    )
    def k(hid_hbm, pf_hbm, hidg_hbm, idx_v, rows_a, rows_b, sem_a, sem_b):
        wid = lax.axis_index("s") * 2 + lax.axis_index("c")
        base0 = wid * (A // NWORK)
        bufs = (rows_a, rows_b)
        sems = (sem_a, sem_b)
        cps = [None] * nch
        for ci in range(nch):
            pltpu.sync_copy(pf_hbm.at[pl.ds(base0 + ci * GCH, GCH)],
                            idx_v.at[ci])
            cps[ci] = pltpu.async_copy(hid_hbm.at[idx_v.at[ci]],
                                       bufs[ci % 2], sems[ci % 2])
            if ci >= 1:
                cps[ci - 1].wait()
                pltpu.sync_copy(
                    bufs[(ci - 1) % 2],
                    hidg_hbm.at[pl.ds(base0 + (ci - 1) * GCH, GCH)])
        cps[nch - 1].wait()
        pltpu.sync_copy(bufs[(nch - 1) % 2],
                        hidg_hbm.at[pl.ds(base0 + (nch - 1) * GCH, GCH)])

    return k(hidw, pf)


OBLK = 512


def _proj_body(h0_ref, h1_ref, wk_ref, wo_ref, bo_ref, out_ref):
    w = wk_ref[...]
    h0l, h0r = _unpack_halves(h0_ref[...])
    h1l, h1r = _unpack_halves(h1_ref[...])
    comb_l = w[:, 0:1] * h0l + w[:, 1:2] * h1l
    comb_r = w[:, 0:1] * h0r + w[:, 1:2] * h1r
    wo = wo_ref[...]
    out_ref[...] = (
        jnp.dot(comb_l, wo[:HW, :], preferred_element_type=jnp.float32)
        + jnp.dot(comb_r, wo[HW:, :], preferred_element_type=jnp.float32)
        + bo_ref[...])


def _proj(hidgw, wk, Wo, bo):
    return pl.pallas_call(
        _proj_body,
        grid=(N // OBLK,),
        in_specs=[
            pl.BlockSpec((OBLK, HW), lambda i: (i, 0)),
            pl.BlockSpec((OBLK, HW), lambda i: (i + N // OBLK, 0)),
            pl.BlockSpec((OBLK, 2), lambda i: (i, 0)),
            _full((H, D)), _full((1, D)),
        ],
        out_specs=pl.BlockSpec((OBLK, D), lambda i: (i, 0)),
        out_shape=jax.ShapeDtypeStruct((N, D), jnp.float32),
    )(hidgw, hidgw, wk, Wo, bo)


@jax.jit
def _run(x2, ft2, W1, b1, W2, b2, W3, b3, type_emb, Wt, bt, We, be3, Wo, bo):
    xw, p, wk, bemap, active = _gate_route(
        x2, ft2, W1, b1, W2, b2, W3, b3, type_emb, Wt, bt)
    p01 = p.T
    xgw = _sc_scatter_rows(xw, p01)
    hidw = _experts(xgw, We, be3, bemap.reshape(NB), active.reshape(NB))
    pf = p01.reshape(A)
    hidgw = _sc_gather_rows(hidw, pf)
    return _proj(hidgw, wk, Wo, bo)


def kernel(x, feature_types, W1, b1, W2, b2, W3, b3, type_emb, Wt, bt, We, be, Wo, bo):
    x2 = x.reshape(N, D)
    ft2 = feature_types.reshape(N, 1).astype(jnp.int32)
    out = _run(x2, ft2, W1, b1.reshape(1, GH), W2, b2.reshape(1, GH // 2),
               W3, b3.reshape(1, E), type_emb, Wt, bt.reshape(1, E),
               We, be.reshape(E, 1, H), Wo, bo.reshape(1, D))
    return out.reshape(B, S, D)
